# baseline (speedup 1.0000x reference)
import functools

import jax
import jax.numpy as jnp
from jax import lax
from jax.experimental import pallas as pl
from jax.experimental.pallas import tpu as pltpu
from jax.experimental.pallas import tpu_sc as plsc

B = 8
T = 4096
D = 512
TT = 4095
NT = 1365
RB = NT * 5
ROWS = B * RB
NW = 32
K = 120
NCHUNK = ROWS // K
FULL_PAIRS = 7
GROUPS = (0, 16, 32, 48, 64, 80, 96, 104)

_mesh = plsc.VectorSubcoreMesh(
    core_axis_name="c", subcore_axis_name="s", num_cores=2, num_subcores=16
)


@functools.partial(
    pl.kernel,
    mesh=_mesh,
    out_type=jax.ShapeDtypeStruct((ROWS, D), jnp.float32),
    scratch_types=[
        pltpu.VMEM((K,), jnp.int32),
        pltpu.VMEM((K,), jnp.int32),
        pltpu.VMEM((K, D), jnp.float32),
        pltpu.VMEM((K, D), jnp.float32),
        pltpu.SemaphoreType.DMA,
        pltpu.SemaphoreType.DMA,
        pltpu.SemaphoreType.DMA,
        pltpu.SemaphoreType.DMA,
    ],
)
def _splice_gather(feats_hbm, out_hbm, idx0_v, idx1_v, rows0_v, rows1_v,
                   sem_g0, sem_g1, sem_o0, sem_o1):
    wid = lax.axis_index("s") * 2 + lax.axis_index("c")
    lanes = lax.iota(jnp.int32, 16)

    def fill_idx(idx_v, cc):
        g0 = cc * K
        b0 = jnp.clip(g0 - RB + 1, 0, 1)
        for bb in range(2, B):
            b0 = b0 + jnp.clip(g0 - bb * RB + 1, 0, 1)
        bound = (b0 + 1) * RB - g0
        for off in GROUPS:
            pos = off + lanes
            b = b0 + jnp.clip(pos - (bound - 1), 0, 1)
            r = g0 + pos - b * RB
            t = (r * 52429) >> 18
            k = r - t * 5
            src = jnp.clip(3 * t + k - 2, 0, TT - 1)
            idx_v[pl.ds(off, 16)] = src + b * T

    def start_gather(idx_v, rows_v, sem, cc):
        fill_idx(idx_v, cc)
        pltpu.async_copy(feats_hbm.at[idx_v], rows_v, sem)

    def wait_gather(idx_v, rows_v, sem):
        pltpu.make_async_copy(feats_hbm.at[idx_v], rows_v, sem).wait()

    def start_put(rows_v, sem, cc):
        pltpu.async_copy(rows_v, out_hbm.at[pl.ds(cc * K, K)], sem)

    def wait_put(rows_v, sem):
        pltpu.make_async_copy(rows_v, out_hbm.at[pl.ds(0, K)], sem).wait()

    def chunk(j):
        return wid + j * NW

    has_extra = wid < NCHUNK - NW * 2 * FULL_PAIRS

    start_gather(idx0_v, rows0_v, sem_g0, chunk(0))

    @pl.loop(0, FULL_PAIRS)
    def _pair(m):
        @pl.when(m > 0)
        def _():
            wait_put(rows1_v, sem_o1)

        start_gather(idx1_v, rows1_v, sem_g1, chunk(2 * m + 1))
        wait_gather(idx0_v, rows0_v, sem_g0)
        start_put(rows0_v, sem_o0, chunk(2 * m))

        @pl.when((m < FULL_PAIRS - 1) | has_extra)
        def _():
            wait_put(rows0_v, sem_o0)
            start_gather(idx0_v, rows0_v, sem_g0, chunk(2 * m + 2))

        wait_gather(idx1_v, rows1_v, sem_g1)
        start_put(rows1_v, sem_o1, chunk(2 * m + 1))

    @pl.when(has_extra)
    def _():
        wait_gather(idx0_v, rows0_v, sem_g0)
        start_put(rows0_v, sem_o0, chunk(2 * FULL_PAIRS))
    wait_put(rows1_v, sem_o1)
    wait_put(rows0_v, sem_o0)


def kernel(feats):
    flat = feats.reshape(B * T, D)
    out = _splice_gather(flat)
    return out.reshape(B, NT, 5 * D)


# speedup vs baseline: 1.1115x; 1.1115x over previous
import functools

import jax
import jax.numpy as jnp
from jax import lax
from jax.experimental import pallas as pl
from jax.experimental.pallas import tpu as pltpu
from jax.experimental.pallas import tpu_sc as plsc

B = 8
T = 4096
D = 512
TT = 4095
NT = 1365
NTB = B * NT
NW = 32
CT = 24
NCHUNK = NTB // CT
FULL_PAIRS = 7
GROUPS = (0, 8)

_mesh = plsc.VectorSubcoreMesh(
    core_axis_name="c", subcore_axis_name="s", num_cores=2, num_subcores=16
)


@functools.partial(
    pl.kernel,
    mesh=_mesh,
    out_type=jax.ShapeDtypeStruct((NTB, 5 * D), jnp.float32),
    scratch_types=[
        pltpu.VMEM((5, CT), jnp.int32),
        pltpu.VMEM((5, CT), jnp.int32),
        pltpu.VMEM((CT, 5 * D), jnp.float32),
        pltpu.VMEM((CT, 5 * D), jnp.float32),
        pltpu.SemaphoreType.DMA,
        pltpu.SemaphoreType.DMA,
        pltpu.SemaphoreType.DMA,
        pltpu.SemaphoreType.DMA,
    ],
)
def _splice_gather(feats_hbm, out_hbm, idx0_v, idx1_v, rows0_v, rows1_v,
                   sem_g0, sem_g1, sem_o0, sem_o1):
    wid = lax.axis_index("s") * 2 + lax.axis_index("c")
    lanes = lax.iota(jnp.int32, 16)

    def fill_idx(idx_v, cc):
        t0 = cc * CT
        b0 = jnp.clip(t0 - NT + 1, 0, 1)
        for bb in range(2, B):
            b0 = b0 + jnp.clip(t0 - bb * NT + 1, 0, 1)
        bound = (b0 + 1) * NT - t0
        for off in GROUPS:
            pos = off + lanes
            b = b0 + jnp.clip(pos - (bound - 1), 0, 1)
            tt = t0 + pos - b * NT
            for kk in range(5):
                idx_v[kk, pl.ds(off, 16)] = (
                    b * T + jnp.clip(3 * tt + (kk - 2), 0, TT - 1))

    def start_gather(idx_v, rows_v, sem, cc):
        fill_idx(idx_v, cc)
        for kk in range(5):
            pltpu.async_copy(feats_hbm.at[idx_v.at[kk]],
                             rows_v.at[:, pl.ds(kk * D, D)], sem)

    def wait_gather(idx_v, rows_v, sem):
        for kk in range(5):
            pltpu.make_async_copy(feats_hbm.at[idx_v.at[kk]],
                                  rows_v.at[:, pl.ds(kk * D, D)], sem).wait()

    def start_put(rows_v, sem, cc):
        pltpu.async_copy(rows_v, out_hbm.at[pl.ds(cc * CT, CT)], sem)

    def wait_put(rows_v, sem):
        pltpu.make_async_copy(rows_v, out_hbm.at[pl.ds(0, CT)], sem).wait()

    def chunk(j):
        return wid + j * NW

    has_extra = wid < NCHUNK - NW * 2 * FULL_PAIRS

    start_gather(idx0_v, rows0_v, sem_g0, chunk(0))

    @pl.loop(0, FULL_PAIRS)
    def _pair(m):
        @pl.when(m > 0)
        def _():
            wait_put(rows1_v, sem_o1)

        start_gather(idx1_v, rows1_v, sem_g1, chunk(2 * m + 1))
        wait_gather(idx0_v, rows0_v, sem_g0)
        start_put(rows0_v, sem_o0, chunk(2 * m))

        @pl.when((m < FULL_PAIRS - 1) | has_extra)
        def _():
            wait_put(rows0_v, sem_o0)
            start_gather(idx0_v, rows0_v, sem_g0, chunk(2 * m + 2))

        wait_gather(idx1_v, rows1_v, sem_g1)
        start_put(rows1_v, sem_o1, chunk(2 * m + 1))

    @pl.when(has_extra)
    def _():
        wait_gather(idx0_v, rows0_v, sem_g0)
        start_put(rows0_v, sem_o0, chunk(2 * FULL_PAIRS))
    wait_put(rows1_v, sem_o1)
    wait_put(rows0_v, sem_o0)


def kernel(feats):
    flat = feats.reshape(B * T, D)
    out = _splice_gather(flat)
    return out.reshape(B, NT, 5 * D)


# speedup vs baseline: 1.6139x; 1.4521x over previous
import functools

import jax
import jax.numpy as jnp
from jax import lax
from jax.experimental import pallas as pl
from jax.experimental.pallas import tpu as pltpu
from jax.experimental.pallas import tpu_sc as plsc

B = 8
T = 4096
D = 512
TT = 4095
NT = 1365
NW = 32
CT = 24
CPB = 56
NFULL = B * CPB
PAIRS = 7
TS = NT - CPB * CT
GROUPS = (0, 8)

_mesh = plsc.VectorSubcoreMesh(
    core_axis_name="c", subcore_axis_name="s", num_cores=2, num_subcores=16
)


@functools.partial(
    pl.kernel,
    mesh=_mesh,
    out_type=(jax.ShapeDtypeStruct((B, NT, 5 * D), jnp.float32),
              jax.ShapeDtypeStruct((B, CT, 5 * D), jnp.float32)),
    scratch_types=[
        pltpu.VMEM((5, CT), jnp.int32),
        pltpu.VMEM((5, CT), jnp.int32),
        pltpu.VMEM((CT, 5 * D), jnp.float32),
        pltpu.VMEM((CT, 5 * D), jnp.float32),
        pltpu.SemaphoreType.DMA,
        pltpu.SemaphoreType.DMA,
        pltpu.SemaphoreType.DMA,
        pltpu.SemaphoreType.DMA,
    ],
)
def _splice_gather(feats_hbm, out_hbm, tail_hbm, idx0_v, idx1_v,
                   rows0_v, rows1_v, sem_g0, sem_g1, sem_o0, sem_o1):
    wid = lax.axis_index("s") * 2 + lax.axis_index("c")
    lanes = lax.iota(jnp.int32, 16)

    def decomp(cc):
        bi = (cc * 1171) >> 16
        return bi, (cc - bi * CPB) * CT

    def fill_idx(idx_v, groups, t0, bi):
        for off in groups:
            tt = t0 + off + lanes
            for kk in range(5):
                idx_v[kk, pl.ds(off, 16)] = (
                    bi * T + jnp.clip(3 * tt + (kk - 2), 0, TT - 1))

    def start_gather(idx_v, rows_v, sem, cc):
        bi, t0 = decomp(cc)
        fill_idx(idx_v, GROUPS, t0, bi)
        for kk in range(5):
            pltpu.async_copy(feats_hbm.at[idx_v.at[kk]],
                             rows_v.at[:, pl.ds(kk * D, D)], sem)

    def wait_gather(idx_v, rows_v, sem):
        for kk in range(5):
            pltpu.make_async_copy(feats_hbm.at[idx_v.at[kk]],
                                  rows_v.at[:, pl.ds(kk * D, D)], sem).wait()

    def start_put(rows_v, sem, cc):
        bi, t0 = decomp(cc)
        pltpu.async_copy(rows_v, out_hbm.at[bi, pl.ds(t0, CT)], sem)

    def wait_put(rows_v, sem):
        pltpu.make_async_copy(rows_v, out_hbm.at[0, pl.ds(0, CT)], sem).wait()

    def chunk(j):
        return wid + j * NW

    start_gather(idx0_v, rows0_v, sem_g0, chunk(0))

    @pl.loop(0, PAIRS)
    def _pair(m):
        @pl.when(m > 0)
        def _():
            wait_put(rows1_v, sem_o1)

        start_gather(idx1_v, rows1_v, sem_g1, chunk(2 * m + 1))
        wait_gather(idx0_v, rows0_v, sem_g0)
        start_put(rows0_v, sem_o0, chunk(2 * m))

        @pl.when(m < PAIRS - 1)
        def _():
            wait_put(rows0_v, sem_o0)
            start_gather(idx0_v, rows0_v, sem_g0, chunk(2 * m + 2))

        wait_gather(idx1_v, rows1_v, sem_g1)
        start_put(rows1_v, sem_o1, chunk(2 * m + 1))

    wait_put(rows1_v, sem_o1)
    wait_put(rows0_v, sem_o0)

    @pl.when(wid < B)
    def _():
        fill_idx(idx0_v, GROUPS, CPB * CT, wid)
        for kk in range(5):
            pltpu.async_copy(feats_hbm.at[idx0_v.at[kk]],
                             rows0_v.at[:, pl.ds(kk * D, D)], sem_g0)
        wait_gather(idx0_v, rows0_v, sem_g0)
        pltpu.async_copy(rows0_v, tail_hbm.at[wid], sem_o0)
        pltpu.make_async_copy(rows0_v, tail_hbm.at[wid], sem_o0).wait()


def kernel(feats):
    out, tail = _splice_gather(feats.reshape(B * T, D))
    return lax.dynamic_update_slice(out, tail[:, :TS], (0, CPB * CT, 0))


# speedup vs baseline: 3.1385x; 1.9447x over previous
import functools

import jax
import jax.numpy as jnp
from jax import lax
from jax.experimental import pallas as pl
from jax.experimental.pallas import tpu as pltpu
from jax.experimental.pallas import tpu_sc as plsc

B = 8
T = 4096
D = 512
TT = 4095
NT = 1365
NW = 32
CTR = 3
NCHUNK = NT // CTR
FULL_PAIRS = 7
KR = CTR * B
GROUPS = (0, 8)

_mesh = plsc.VectorSubcoreMesh(
    core_axis_name="c", subcore_axis_name="s", num_cores=2, num_subcores=16
)


@functools.partial(
    pl.kernel,
    mesh=_mesh,
    out_type=jax.ShapeDtypeStruct((NT, B, 5 * D), jnp.float32),
    scratch_types=[
        pltpu.VMEM((5, KR), jnp.int32),
        pltpu.VMEM((5, KR), jnp.int32),
        pltpu.VMEM((KR, 5 * D), jnp.float32),
        pltpu.VMEM((KR, 5 * D), jnp.float32),
        pltpu.SemaphoreType.DMA,
        pltpu.SemaphoreType.DMA,
        pltpu.SemaphoreType.DMA,
        pltpu.SemaphoreType.DMA,
    ],
)
def _splice_gather(feats_hbm, out_hbm, idx0_v, idx1_v, rows0_v, rows1_v,
                   sem_g0, sem_g1, sem_o0, sem_o1):
    wid = lax.axis_index("s") * 2 + lax.axis_index("c")
    lanes = lax.iota(jnp.int32, 16)

    pats = []
    for off in GROUPS:
        pos = off + lanes
        tloc = lax.shift_right_logical(pos, 3)
        bpat = (pos - tloc * B) * T
        pats.append((3 * tloc, bpat))

    def fill_idx(idx_v, cc):
        s0 = cc * (3 * CTR)
        for (t3, bpat), off in zip(pats, GROUPS):
            for kk in range(5):
                idx_v[kk, pl.ds(off, 16)] = (
                    bpat + jnp.clip(s0 + t3 + (kk - 2), 0, TT - 1))

    def start_gather(idx_v, rows_v, sem, cc):
        fill_idx(idx_v, cc)
        for kk in range(5):
            pltpu.async_copy(feats_hbm.at[idx_v.at[kk]],
                             rows_v.at[:, pl.ds(kk * D, D)], sem)

    def wait_gather(idx_v, rows_v, sem):
        for kk in range(5):
            pltpu.make_async_copy(feats_hbm.at[idx_v.at[kk]],
                                  rows_v.at[:, pl.ds(kk * D, D)], sem).wait()

    def start_put(rows_v, sem, cc):
        pltpu.async_copy(rows_v.reshape(CTR, B, 5 * D),
                         out_hbm.at[pl.ds(cc * CTR, CTR)], sem)

    def wait_put(rows_v, sem):
        pltpu.make_async_copy(rows_v.reshape(CTR, B, 5 * D),
                              out_hbm.at[pl.ds(0, CTR)], sem).wait()

    def chunk(j):
        return wid + j * NW

    has_extra = wid < NCHUNK - NW * 2 * FULL_PAIRS

    start_gather(idx0_v, rows0_v, sem_g0, chunk(0))

    @pl.loop(0, FULL_PAIRS)
    def _pair(m):
        @pl.when(m > 0)
        def _():
            wait_put(rows1_v, sem_o1)

        start_gather(idx1_v, rows1_v, sem_g1, chunk(2 * m + 1))
        wait_gather(idx0_v, rows0_v, sem_g0)
        start_put(rows0_v, sem_o0, chunk(2 * m))

        @pl.when((m < FULL_PAIRS - 1) | has_extra)
        def _():
            wait_put(rows0_v, sem_o0)
            start_gather(idx0_v, rows0_v, sem_g0, chunk(2 * m + 2))

        wait_gather(idx1_v, rows1_v, sem_g1)
        start_put(rows1_v, sem_o1, chunk(2 * m + 1))

    @pl.when(has_extra)
    def _():
        wait_gather(idx0_v, rows0_v, sem_g0)
        start_put(rows0_v, sem_o0, chunk(2 * FULL_PAIRS))
    wait_put(rows1_v, sem_o1)
    wait_put(rows0_v, sem_o0)


def kernel(feats):
    out = _splice_gather(feats.reshape(B * T, D))
    return out.transpose(1, 0, 2)
